# Initial kernel scaffold; baseline (speedup 1.0000x reference)
import jax
import jax.numpy as jnp
from jax.experimental import pallas as pl


def kernel(positions, viewdirs, embedded_appearance, embedded_transient, tables, W1, W2):
    raise NotImplementedError("write your pallas kernel here")


# speedup vs baseline: 224.0227x; 224.0227x over previous
import functools

import jax
import jax.numpy as jnp
import numpy as np
from jax import lax
from jax.experimental import pallas as pl
from jax.experimental.pallas import tpu as pltpu
from jax.experimental.pallas import tpu_sc as plsc

NUM_LEVELS = 8
BASE_RES = 16
MAX_RES = 1024
LOG2_T = 18
T = 2 ** LOG2_T
BOUND = 2.0
N_POINTS = 1048576
HIDDEN = 64
GROWTH = np.exp((np.log(MAX_RES) - np.log(BASE_RES)) / (NUM_LEVELS - 1))
RESOLUTIONS = [float(np.floor(BASE_RES * GROWTH ** l)) for l in range(NUM_LEVELS)]
P1_I32 = np.int32(np.uint32(2654435761).view(np.int32))
P2_I32 = np.int32(np.uint32(805459861).view(np.int32))

R = 1024
CB = 1024
SB = 8
NBLK = R // SB
PTS_PER_BLK = SB * CB
NIDX = NUM_LEVELS * 8 * N_POINTS

SC_WORKERS = 32
PER_W = NIDX // SC_WORKERS
CHUNK = 8192
NCHUNK = PER_W // CHUNK


def _hash_body(xs_ref, ys_ref, zs_ref, idx_ref, sel_ref):
    x = xs_ref[...]
    y = ys_ref[...]
    z = zs_ref[...]
    inv = 1.0 / (2.0 * BOUND)
    px = (x + BOUND) * inv
    py = (y + BOUND) * inv
    pz = (z + BOUND) * inv
    sel = ((px >= 0.0) & (px <= 1.0) & (py >= 0.0) & (py <= 1.0)
           & (pz >= 0.0) & (pz <= 1.0))
    fsel = sel.astype(jnp.float32)
    px = px * fsel
    py = py * fsel
    pz = pz * fsel
    for l in range(NUM_LEVELS):
        res = np.float32(RESOLUTIONS[l])
        xi = jnp.floor(px * res).astype(jnp.int32)
        yi = jnp.floor(py * res).astype(jnp.int32)
        zi = jnp.floor(pz * res).astype(jnp.int32)
        hx = (xi, xi + 1)
        hy0 = yi * P1_I32
        hy = (hy0, hy0 + P1_I32)
        hz0 = zi * P2_I32
        hz = (hz0, hz0 + P2_I32)
        base = np.int32(l * T)
        for c in range(8):
            h = hx[c & 1] ^ hy[(c >> 1) & 1] ^ hz[(c >> 2) & 1]
            idx_ref[l * 8 + c] = (h & np.int32(T - 1)) | base
    sel_ref[...] = fsel


def _gather_kernel_body(pt_ref, idx_ref, g_ref, idx_v, g_v, sem):
    wid = lax.axis_index("s") * 2 + lax.axis_index("c")
    base = wid * PER_W

    @pl.loop(0, NCHUNK)
    def _chunk(j):
        off = pl.multiple_of(base + j * CHUNK, 8)
        pltpu.sync_copy(idx_ref.at[pl.ds(off, CHUNK)], idx_v)
        pltpu.async_copy(pt_ref.at[idx_v], g_v, sem).wait()
        pltpu.sync_copy(g_v, g_ref.at[pl.ds(off, CHUNK)])


def _blend_body(xs_ref, ys_ref, zs_ref, g_ref, feats_ref):
    x = xs_ref[...]
    y = ys_ref[...]
    z = zs_ref[...]
    inv = 1.0 / (2.0 * BOUND)
    px = (x + BOUND) * inv
    py = (y + BOUND) * inv
    pz = (z + BOUND) * inv
    sel = ((px >= 0.0) & (px <= 1.0) & (py >= 0.0) & (py <= 1.0)
           & (pz >= 0.0) & (pz <= 1.0))
    fsel = sel.astype(jnp.float32)
    px = px * fsel
    py = py * fsel
    pz = pz * fsel
    for l in range(NUM_LEVELS):
        res = np.float32(RESOLUTIONS[l])
        fx = px * res
        fy = py * res
        fz = pz * res
        wx = fx - jnp.floor(fx)
        wy = fy - jnp.floor(fy)
        wz = fz - jnp.floor(fz)
        wxs = (1.0 - wx, wx)
        wys = (1.0 - wy, wy)
        wzs = (1.0 - wz, wz)
        f0 = jnp.zeros((SB, CB), jnp.float32)
        f1 = jnp.zeros((SB, CB), jnp.float32)
        for cz in range(2):
            for cy in range(2):
                wyz = wys[cy] * wzs[cz]
                for cx in range(2):
                    c = cx | (cy << 1) | (cz << 2)
                    g = g_ref[l * 8 + c]
                    a0 = lax.bitcast_convert_type(g << 16, jnp.float32)
                    a1 = lax.bitcast_convert_type(g & np.int32(-65536),
                                                  jnp.float32)
                    wc = wxs[cx] * wyz
                    f0 = f0 + wc * a0
                    f1 = f1 + wc * a1
        feats_ref[2 * l] = f0
        feats_ref[2 * l + 1] = f1


def _mlp_body(f_ref, sel_ref, w1t_ref, w2_ref, out_ref):
    fb = f_ref[...].astype(jnp.bfloat16)
    w1t = w1t_ref[...]
    h = lax.dot_general(w1t, fb, (((1,), (0,)), ((), ())),
                        preferred_element_type=jnp.float32)
    h = jnp.maximum(h, 0.0).astype(jnp.bfloat16)
    w2 = w2_ref[...]
    raw = lax.dot_general(w2, h, (((1,), (0,)), ((), ())),
                          preferred_element_type=jnp.float32)
    out_ref[...] = (jnp.exp(raw) * sel_ref[0])[None]


def _hash_call(xs, ys, zs):
    return pl.pallas_call(
        _hash_body,
        grid=(NBLK,),
        in_specs=[
            pl.BlockSpec((SB, CB), lambda i: (i, 0)),
            pl.BlockSpec((SB, CB), lambda i: (i, 0)),
            pl.BlockSpec((SB, CB), lambda i: (i, 0)),
        ],
        out_specs=[
            pl.BlockSpec((NUM_LEVELS * 8, SB, CB), lambda i: (0, i, 0)),
            pl.BlockSpec((SB, CB), lambda i: (i, 0)),
        ],
        out_shape=[
            jax.ShapeDtypeStruct((NUM_LEVELS * 8, R, CB), jnp.int32),
            jax.ShapeDtypeStruct((R, CB), jnp.float32),
        ],
    )(xs, ys, zs)


def _gather_call(pt_flat, idx_flat):
    mesh = plsc.VectorSubcoreMesh(core_axis_name="c", subcore_axis_name="s")
    kern = functools.partial(
        pl.kernel,
        out_type=jax.ShapeDtypeStruct((NIDX,), jnp.int32),
        mesh=mesh,
        scratch_types=[
            pltpu.VMEM((CHUNK,), jnp.int32),
            pltpu.VMEM((CHUNK,), jnp.int32),
            pltpu.SemaphoreType.DMA,
        ],
    )(_gather_kernel_body)
    return kern(pt_flat, idx_flat)


def _blend_call(xs, ys, zs, g):
    return pl.pallas_call(
        _blend_body,
        grid=(NBLK,),
        in_specs=[
            pl.BlockSpec((SB, CB), lambda i: (i, 0)),
            pl.BlockSpec((SB, CB), lambda i: (i, 0)),
            pl.BlockSpec((SB, CB), lambda i: (i, 0)),
            pl.BlockSpec((NUM_LEVELS * 8, SB, CB), lambda i: (0, i, 0)),
        ],
        out_specs=pl.BlockSpec((2 * NUM_LEVELS, SB, CB), lambda i: (0, i, 0)),
        out_shape=jax.ShapeDtypeStruct((2 * NUM_LEVELS, R, CB), jnp.float32),
    )(xs, ys, zs, g)


def _mlp_call(feats2d, sel2d, w1t, w2r):
    return pl.pallas_call(
        _mlp_body,
        grid=(NBLK,),
        in_specs=[
            pl.BlockSpec((2 * NUM_LEVELS, PTS_PER_BLK), lambda i: (0, i)),
            pl.BlockSpec((1, 1, PTS_PER_BLK), lambda i: (i, 0, 0)),
            pl.BlockSpec((HIDDEN, 2 * NUM_LEVELS), lambda i: (0, 0)),
            pl.BlockSpec((1, HIDDEN), lambda i: (0, 0)),
        ],
        out_specs=pl.BlockSpec((1, 1, PTS_PER_BLK), lambda i: (i, 0, 0)),
        out_shape=jax.ShapeDtypeStruct((NBLK, 1, PTS_PER_BLK), jnp.float32),
    )(feats2d, sel2d, w1t, w2r)


def kernel(positions, viewdirs, embedded_appearance, embedded_transient,
           tables, W1, W2):
    pos_t = positions.T.reshape(3, R, CB)
    xs, ys, zs = pos_t[0], pos_t[1], pos_t[2]
    tb = lax.bitcast_convert_type(tables.astype(jnp.bfloat16), jnp.uint16)
    pt = (tb[..., 0].astype(jnp.uint32)
          | (tb[..., 1].astype(jnp.uint32) << 16))
    pt_flat = lax.bitcast_convert_type(pt, jnp.int32).reshape(NUM_LEVELS * T)
    w1t = W1.T.astype(jnp.bfloat16)
    w2r = W2.reshape(1, HIDDEN).astype(jnp.bfloat16)

    idx, fsel = _hash_call(xs, ys, zs)
    g_flat = _gather_call(pt_flat, idx.reshape(NIDX))
    g = g_flat.reshape(NUM_LEVELS * 8, R, CB)
    feats = _blend_call(xs, ys, zs, g)
    feats2d = feats.reshape(2 * NUM_LEVELS, N_POINTS)
    sel2d = fsel.reshape(NBLK, 1, PTS_PER_BLK)
    out = _mlp_call(feats2d, sel2d, w1t, w2r)
    return out.reshape(N_POINTS, 1)


# speedup vs baseline: 575.2807x; 2.5680x over previous
import functools

import jax
import jax.numpy as jnp
import numpy as np
from jax import lax
from jax.experimental import pallas as pl
from jax.experimental.pallas import tpu as pltpu
from jax.experimental.pallas import tpu_sc as plsc

NUM_LEVELS = 8
BASE_RES = 16
MAX_RES = 1024
LOG2_T = 18
T = 2 ** LOG2_T
BOUND = 2.0
N_POINTS = 1048576
HIDDEN = 64
GROWTH = np.exp((np.log(MAX_RES) - np.log(BASE_RES)) / (NUM_LEVELS - 1))
RESOLUTIONS = [float(np.floor(BASE_RES * GROWTH ** l)) for l in range(NUM_LEVELS)]
P1_I32 = np.int32(np.uint32(2654435761).view(np.int32))
P2_I32 = np.int32(np.uint32(805459861).view(np.int32))

R = 1024
CB = 1024
SB = 8
NBLK = R // SB
PTS_PER_BLK = SB * CB
NIDX = NUM_LEVELS * 8 * N_POINTS

SC_WORKERS = 32
PER_W = NIDX // SC_WORKERS
CHUNK = 16384
NCHUNK = PER_W // CHUNK
HALF_T = 4 * T
STAGE_W = HALF_T // 16


def _hash_body(xs_ref, ys_ref, zs_ref, idx_ref, sel_ref):
    x = xs_ref[...]
    y = ys_ref[...]
    z = zs_ref[...]
    inv = 1.0 / (2.0 * BOUND)
    px = (x + BOUND) * inv
    py = (y + BOUND) * inv
    pz = (z + BOUND) * inv
    sel = ((px >= 0.0) & (px <= 1.0) & (py >= 0.0) & (py <= 1.0)
           & (pz >= 0.0) & (pz <= 1.0))
    fsel = sel.astype(jnp.float32)
    px = px * fsel
    py = py * fsel
    pz = pz * fsel
    for l in range(NUM_LEVELS):
        res = np.float32(RESOLUTIONS[l])
        xi = jnp.floor(px * res).astype(jnp.int32)
        yi = jnp.floor(py * res).astype(jnp.int32)
        zi = jnp.floor(pz * res).astype(jnp.int32)
        hx = (xi, xi + 1)
        hy0 = yi * P1_I32
        hy = (hy0, hy0 + P1_I32)
        hz0 = zi * P2_I32
        hz = (hz0, hz0 + P2_I32)
        base = np.int32((l % 4) * T)
        for c in range(8):
            h = hx[c & 1] ^ hy[(c >> 1) & 1] ^ hz[(c >> 2) & 1]
            idx_ref[l * 8 + c] = (h & np.int32(T - 1)) | base
    sel_ref[...] = fsel


def _gather_kernel_body(pt_ref, idx_ref, g_ref, tab_sp, idx_v, g_v, sem):
    cid = lax.axis_index("c")
    sid = lax.axis_index("s")
    so = pl.multiple_of(sid * STAGE_W, 8)
    src = pl.multiple_of(cid * HALF_T + so, 8)
    pltpu.sync_copy(pt_ref.at[pl.ds(src, STAGE_W)], tab_sp.at[pl.ds(so, STAGE_W)])
    plsc.subcore_barrier()

    base = (cid * 16 + sid) * PER_W

    @pl.loop(0, NCHUNK)
    def _chunk(j):
        off = pl.multiple_of(base + j * CHUNK, 8)
        pltpu.sync_copy(idx_ref.at[pl.ds(off, CHUNK)], idx_v)
        pltpu.async_copy(tab_sp.at[idx_v], g_v, sem).wait()
        pltpu.sync_copy(g_v, g_ref.at[pl.ds(off, CHUNK)])


def _blend_body(xs_ref, ys_ref, zs_ref, g_ref, feats_ref):
    x = xs_ref[...]
    y = ys_ref[...]
    z = zs_ref[...]
    inv = 1.0 / (2.0 * BOUND)
    px = (x + BOUND) * inv
    py = (y + BOUND) * inv
    pz = (z + BOUND) * inv
    sel = ((px >= 0.0) & (px <= 1.0) & (py >= 0.0) & (py <= 1.0)
           & (pz >= 0.0) & (pz <= 1.0))
    fsel = sel.astype(jnp.float32)
    px = px * fsel
    py = py * fsel
    pz = pz * fsel
    for l in range(NUM_LEVELS):
        res = np.float32(RESOLUTIONS[l])
        fx = px * res
        fy = py * res
        fz = pz * res
        wx = fx - jnp.floor(fx)
        wy = fy - jnp.floor(fy)
        wz = fz - jnp.floor(fz)
        wxs = (1.0 - wx, wx)
        wys = (1.0 - wy, wy)
        wzs = (1.0 - wz, wz)
        f0 = jnp.zeros((SB, CB), jnp.float32)
        f1 = jnp.zeros((SB, CB), jnp.float32)
        for cz in range(2):
            for cy in range(2):
                wyz = wys[cy] * wzs[cz]
                for cx in range(2):
                    c = cx | (cy << 1) | (cz << 2)
                    g = g_ref[l * 8 + c]
                    a0 = lax.bitcast_convert_type(g << 16, jnp.float32)
                    a1 = lax.bitcast_convert_type(g & np.int32(-65536),
                                                  jnp.float32)
                    wc = wxs[cx] * wyz
                    f0 = f0 + wc * a0
                    f1 = f1 + wc * a1
        feats_ref[2 * l] = f0
        feats_ref[2 * l + 1] = f1


def _mlp_body(f_ref, sel_ref, w1t_ref, w2_ref, out_ref):
    fb = f_ref[...].astype(jnp.bfloat16)
    w1t = w1t_ref[...]
    h = lax.dot_general(w1t, fb, (((1,), (0,)), ((), ())),
                        preferred_element_type=jnp.float32)
    h = jnp.maximum(h, 0.0).astype(jnp.bfloat16)
    w2 = w2_ref[...]
    raw = lax.dot_general(w2, h, (((1,), (0,)), ((), ())),
                          preferred_element_type=jnp.float32)
    out_ref[...] = (jnp.exp(raw) * sel_ref[0])[None]


def _hash_call(xs, ys, zs):
    return pl.pallas_call(
        _hash_body,
        grid=(NBLK,),
        in_specs=[
            pl.BlockSpec((SB, CB), lambda i: (i, 0)),
            pl.BlockSpec((SB, CB), lambda i: (i, 0)),
            pl.BlockSpec((SB, CB), lambda i: (i, 0)),
        ],
        out_specs=[
            pl.BlockSpec((NUM_LEVELS * 8, SB, CB), lambda i: (0, i, 0)),
            pl.BlockSpec((SB, CB), lambda i: (i, 0)),
        ],
        out_shape=[
            jax.ShapeDtypeStruct((NUM_LEVELS * 8, R, CB), jnp.int32),
            jax.ShapeDtypeStruct((R, CB), jnp.float32),
        ],
    )(xs, ys, zs)


def _gather_call(pt_flat, idx_flat):
    mesh = plsc.VectorSubcoreMesh(core_axis_name="c", subcore_axis_name="s")
    kern = functools.partial(
        pl.kernel,
        out_type=jax.ShapeDtypeStruct((NIDX,), jnp.int32),
        mesh=mesh,
        scratch_types=[
            pltpu.VMEM_SHARED((HALF_T,), jnp.int32),
            pltpu.VMEM((CHUNK,), jnp.int32),
            pltpu.VMEM((CHUNK,), jnp.int32),
            pltpu.SemaphoreType.DMA,
        ],
    )(_gather_kernel_body)
    return kern(pt_flat, idx_flat)


def _blend_call(xs, ys, zs, g):
    return pl.pallas_call(
        _blend_body,
        grid=(NBLK,),
        in_specs=[
            pl.BlockSpec((SB, CB), lambda i: (i, 0)),
            pl.BlockSpec((SB, CB), lambda i: (i, 0)),
            pl.BlockSpec((SB, CB), lambda i: (i, 0)),
            pl.BlockSpec((NUM_LEVELS * 8, SB, CB), lambda i: (0, i, 0)),
        ],
        out_specs=pl.BlockSpec((2 * NUM_LEVELS, SB, CB), lambda i: (0, i, 0)),
        out_shape=jax.ShapeDtypeStruct((2 * NUM_LEVELS, R, CB), jnp.float32),
    )(xs, ys, zs, g)


def _mlp_call(feats2d, sel2d, w1t, w2r):
    return pl.pallas_call(
        _mlp_body,
        grid=(NBLK,),
        in_specs=[
            pl.BlockSpec((2 * NUM_LEVELS, PTS_PER_BLK), lambda i: (0, i)),
            pl.BlockSpec((1, 1, PTS_PER_BLK), lambda i: (i, 0, 0)),
            pl.BlockSpec((HIDDEN, 2 * NUM_LEVELS), lambda i: (0, 0)),
            pl.BlockSpec((1, HIDDEN), lambda i: (0, 0)),
        ],
        out_specs=pl.BlockSpec((1, 1, PTS_PER_BLK), lambda i: (i, 0, 0)),
        out_shape=jax.ShapeDtypeStruct((NBLK, 1, PTS_PER_BLK), jnp.float32),
    )(feats2d, sel2d, w1t, w2r)


def kernel(positions, viewdirs, embedded_appearance, embedded_transient,
           tables, W1, W2):
    pos_t = positions.T.reshape(3, R, CB)
    xs, ys, zs = pos_t[0], pos_t[1], pos_t[2]
    tb = lax.bitcast_convert_type(tables.astype(jnp.bfloat16), jnp.uint16)
    pt = (tb[..., 0].astype(jnp.uint32)
          | (tb[..., 1].astype(jnp.uint32) << 16))
    pt_flat = lax.bitcast_convert_type(pt, jnp.int32).reshape(NUM_LEVELS * T)
    w1t = W1.T.astype(jnp.bfloat16)
    w2r = W2.reshape(1, HIDDEN).astype(jnp.bfloat16)

    idx, fsel = _hash_call(xs, ys, zs)
    g_flat = _gather_call(pt_flat, idx.reshape(NIDX))
    g = g_flat.reshape(NUM_LEVELS * 8, R, CB)
    feats = _blend_call(xs, ys, zs, g)
    feats2d = feats.reshape(2 * NUM_LEVELS, N_POINTS)
    sel2d = fsel.reshape(NBLK, 1, PTS_PER_BLK)
    out = _mlp_call(feats2d, sel2d, w1t, w2r)
    return out.reshape(N_POINTS, 1)


# speedup vs baseline: 601.3300x; 1.0453x over previous
import functools

import jax
import jax.numpy as jnp
import numpy as np
from jax import lax
from jax.experimental import pallas as pl
from jax.experimental.pallas import tpu as pltpu
from jax.experimental.pallas import tpu_sc as plsc

NUM_LEVELS = 8
BASE_RES = 16
MAX_RES = 1024
LOG2_T = 18
T = 2 ** LOG2_T
BOUND = 2.0
N_POINTS = 1048576
HIDDEN = 64
GROWTH = np.exp((np.log(MAX_RES) - np.log(BASE_RES)) / (NUM_LEVELS - 1))
RESOLUTIONS = [float(np.floor(BASE_RES * GROWTH ** l)) for l in range(NUM_LEVELS)]
P1_I32 = np.int32(np.uint32(2654435761).view(np.int32))
P2_I32 = np.int32(np.uint32(805459861).view(np.int32))

R = 1024
CB = 1024
SB = 8
NBLK = R // SB
PTS_PER_BLK = SB * CB
NIDX = NUM_LEVELS * 8 * N_POINTS

SC_WORKERS = 32
PER_W = NIDX // SC_WORKERS
CHUNK = 16384
NCHUNK = PER_W // CHUNK
HALF_T = 4 * T
STAGE_W = HALF_T // 16


def _hash_body(xs_ref, ys_ref, zs_ref, idx_ref, sel_ref):
    x = xs_ref[...]
    y = ys_ref[...]
    z = zs_ref[...]
    inv = 1.0 / (2.0 * BOUND)
    px = (x + BOUND) * inv
    py = (y + BOUND) * inv
    pz = (z + BOUND) * inv
    sel = ((px >= 0.0) & (px <= 1.0) & (py >= 0.0) & (py <= 1.0)
           & (pz >= 0.0) & (pz <= 1.0))
    fsel = sel.astype(jnp.float32)
    px = px * fsel
    py = py * fsel
    pz = pz * fsel
    for l in range(NUM_LEVELS):
        res = np.float32(RESOLUTIONS[l])
        xi = jnp.floor(px * res).astype(jnp.int32)
        yi = jnp.floor(py * res).astype(jnp.int32)
        zi = jnp.floor(pz * res).astype(jnp.int32)
        hx = (xi, xi + 1)
        hy0 = yi * P1_I32
        hy = (hy0, hy0 + P1_I32)
        hz0 = zi * P2_I32
        hz = (hz0, hz0 + P2_I32)
        base = np.int32((l % 4) * T)
        for c in range(8):
            h = hx[c & 1] ^ hy[(c >> 1) & 1] ^ hz[(c >> 2) & 1]
            idx_ref[l * 8 + c] = (h & np.int32(T - 1)) | base
    sel_ref[...] = fsel


def _gather_kernel_body(pt_ref, idx_ref, g_ref, tab_sp,
                        idx_v0, idx_v1, g_v0, g_v1,
                        si0, si1, sg0, sg1, so0, so1):
    cid = lax.axis_index("c")
    sid = lax.axis_index("s")
    so = pl.multiple_of(sid * STAGE_W, 8)
    src = pl.multiple_of(cid * HALF_T + so, 8)
    pltpu.sync_copy(pt_ref.at[pl.ds(src, STAGE_W)], tab_sp.at[pl.ds(so, STAGE_W)])
    plsc.subcore_barrier()

    base = (cid * 16 + sid) * PER_W
    npair = NCHUNK // 2

    def offs(jj):
        o0 = pl.multiple_of(base + (2 * jj) * CHUNK, 8)
        return o0, pl.multiple_of(base + (2 * jj + 1) * CHUNK, 8)

    o0, o1 = offs(0)
    pltpu.async_copy(idx_ref.at[pl.ds(o0, CHUNK)], idx_v0, si0)
    pltpu.async_copy(idx_ref.at[pl.ds(o1, CHUNK)], idx_v1, si1)

    @pl.loop(0, npair)
    def _pair(jj):
        o0, o1 = offs(jj)

        @pl.when(jj > 0)
        def _():
            pltpu.make_async_copy(g_v0, g_ref.at[pl.ds(o0, CHUNK)], so0).wait()
            pltpu.make_async_copy(g_v1, g_ref.at[pl.ds(o1, CHUNK)], so1).wait()

        pltpu.make_async_copy(idx_ref.at[pl.ds(o0, CHUNK)], idx_v0, si0).wait()
        pltpu.async_copy(tab_sp.at[idx_v0], g_v0, sg0)
        pltpu.make_async_copy(idx_ref.at[pl.ds(o1, CHUNK)], idx_v1, si1).wait()
        pltpu.async_copy(tab_sp.at[idx_v1], g_v1, sg1)

        pltpu.make_async_copy(tab_sp.at[idx_v0], g_v0, sg0).wait()
        pltpu.async_copy(g_v0, g_ref.at[pl.ds(o0, CHUNK)], so0)
        pltpu.make_async_copy(tab_sp.at[idx_v1], g_v1, sg1).wait()
        pltpu.async_copy(g_v1, g_ref.at[pl.ds(o1, CHUNK)], so1)

        @pl.when(jj < npair - 1)
        def _():
            n0 = pl.multiple_of(base + (2 * jj + 2) * CHUNK, 8)
            n1 = pl.multiple_of(base + (2 * jj + 3) * CHUNK, 8)
            pltpu.async_copy(idx_ref.at[pl.ds(n0, CHUNK)], idx_v0, si0)
            pltpu.async_copy(idx_ref.at[pl.ds(n1, CHUNK)], idx_v1, si1)

    oL0, oL1 = offs(npair - 1)
    pltpu.make_async_copy(g_v0, g_ref.at[pl.ds(oL0, CHUNK)], so0).wait()
    pltpu.make_async_copy(g_v1, g_ref.at[pl.ds(oL1, CHUNK)], so1).wait()


def _blend_body(xs_ref, ys_ref, zs_ref, g_ref, feats_ref):
    x = xs_ref[...]
    y = ys_ref[...]
    z = zs_ref[...]
    inv = 1.0 / (2.0 * BOUND)
    px = (x + BOUND) * inv
    py = (y + BOUND) * inv
    pz = (z + BOUND) * inv
    sel = ((px >= 0.0) & (px <= 1.0) & (py >= 0.0) & (py <= 1.0)
           & (pz >= 0.0) & (pz <= 1.0))
    fsel = sel.astype(jnp.float32)
    px = px * fsel
    py = py * fsel
    pz = pz * fsel
    for l in range(NUM_LEVELS):
        res = np.float32(RESOLUTIONS[l])
        fx = px * res
        fy = py * res
        fz = pz * res
        wx = fx - jnp.floor(fx)
        wy = fy - jnp.floor(fy)
        wz = fz - jnp.floor(fz)
        wxs = (1.0 - wx, wx)
        wys = (1.0 - wy, wy)
        wzs = (1.0 - wz, wz)
        f0 = jnp.zeros((SB, CB), jnp.float32)
        f1 = jnp.zeros((SB, CB), jnp.float32)
        for cz in range(2):
            for cy in range(2):
                wyz = wys[cy] * wzs[cz]
                for cx in range(2):
                    c = cx | (cy << 1) | (cz << 2)
                    g = g_ref[l * 8 + c]
                    a0 = lax.bitcast_convert_type(g << 16, jnp.float32)
                    a1 = lax.bitcast_convert_type(g & np.int32(-65536),
                                                  jnp.float32)
                    wc = wxs[cx] * wyz
                    f0 = f0 + wc * a0
                    f1 = f1 + wc * a1
        feats_ref[2 * l] = f0
        feats_ref[2 * l + 1] = f1


def _mlp_body(f_ref, sel_ref, w1t_ref, w2_ref, out_ref):
    fb = f_ref[...].astype(jnp.bfloat16)
    w1t = w1t_ref[...]
    h = lax.dot_general(w1t, fb, (((1,), (0,)), ((), ())),
                        preferred_element_type=jnp.float32)
    h = jnp.maximum(h, 0.0).astype(jnp.bfloat16)
    w2 = w2_ref[...]
    raw = lax.dot_general(w2, h, (((1,), (0,)), ((), ())),
                          preferred_element_type=jnp.float32)
    out_ref[...] = (jnp.exp(raw) * sel_ref[0])[None]


def _hash_call(xs, ys, zs):
    return pl.pallas_call(
        _hash_body,
        grid=(NBLK,),
        in_specs=[
            pl.BlockSpec((SB, CB), lambda i: (i, 0)),
            pl.BlockSpec((SB, CB), lambda i: (i, 0)),
            pl.BlockSpec((SB, CB), lambda i: (i, 0)),
        ],
        out_specs=[
            pl.BlockSpec((NUM_LEVELS * 8, SB, CB), lambda i: (0, i, 0)),
            pl.BlockSpec((SB, CB), lambda i: (i, 0)),
        ],
        out_shape=[
            jax.ShapeDtypeStruct((NUM_LEVELS * 8, R, CB), jnp.int32),
            jax.ShapeDtypeStruct((R, CB), jnp.float32),
        ],
    )(xs, ys, zs)


def _gather_call(pt_flat, idx_flat):
    mesh = plsc.VectorSubcoreMesh(core_axis_name="c", subcore_axis_name="s")
    kern = functools.partial(
        pl.kernel,
        out_type=jax.ShapeDtypeStruct((NIDX,), jnp.int32),
        mesh=mesh,
        scratch_types=[
            pltpu.VMEM_SHARED((HALF_T,), jnp.int32),
            pltpu.VMEM((CHUNK,), jnp.int32),
            pltpu.VMEM((CHUNK,), jnp.int32),
            pltpu.VMEM((CHUNK,), jnp.int32),
            pltpu.VMEM((CHUNK,), jnp.int32),
            pltpu.SemaphoreType.DMA,
            pltpu.SemaphoreType.DMA,
            pltpu.SemaphoreType.DMA,
            pltpu.SemaphoreType.DMA,
            pltpu.SemaphoreType.DMA,
            pltpu.SemaphoreType.DMA,
        ],
    )(_gather_kernel_body)
    return kern(pt_flat, idx_flat)


def _blend_call(xs, ys, zs, g):
    return pl.pallas_call(
        _blend_body,
        grid=(NBLK,),
        in_specs=[
            pl.BlockSpec((SB, CB), lambda i: (i, 0)),
            pl.BlockSpec((SB, CB), lambda i: (i, 0)),
            pl.BlockSpec((SB, CB), lambda i: (i, 0)),
            pl.BlockSpec((NUM_LEVELS * 8, SB, CB), lambda i: (0, i, 0)),
        ],
        out_specs=pl.BlockSpec((2 * NUM_LEVELS, SB, CB), lambda i: (0, i, 0)),
        out_shape=jax.ShapeDtypeStruct((2 * NUM_LEVELS, R, CB), jnp.float32),
    )(xs, ys, zs, g)


def _mlp_call(feats2d, sel2d, w1t, w2r):
    return pl.pallas_call(
        _mlp_body,
        grid=(NBLK,),
        in_specs=[
            pl.BlockSpec((2 * NUM_LEVELS, PTS_PER_BLK), lambda i: (0, i)),
            pl.BlockSpec((1, 1, PTS_PER_BLK), lambda i: (i, 0, 0)),
            pl.BlockSpec((HIDDEN, 2 * NUM_LEVELS), lambda i: (0, 0)),
            pl.BlockSpec((1, HIDDEN), lambda i: (0, 0)),
        ],
        out_specs=pl.BlockSpec((1, 1, PTS_PER_BLK), lambda i: (i, 0, 0)),
        out_shape=jax.ShapeDtypeStruct((NBLK, 1, PTS_PER_BLK), jnp.float32),
    )(feats2d, sel2d, w1t, w2r)


def kernel(positions, viewdirs, embedded_appearance, embedded_transient,
           tables, W1, W2):
    pos_t = positions.T.reshape(3, R, CB)
    xs, ys, zs = pos_t[0], pos_t[1], pos_t[2]
    tb = lax.bitcast_convert_type(tables.astype(jnp.bfloat16), jnp.uint16)
    pt = (tb[..., 0].astype(jnp.uint32)
          | (tb[..., 1].astype(jnp.uint32) << 16))
    pt_flat = lax.bitcast_convert_type(pt, jnp.int32).reshape(NUM_LEVELS * T)
    w1t = W1.T.astype(jnp.bfloat16)
    w2r = W2.reshape(1, HIDDEN).astype(jnp.bfloat16)

    idx, fsel = _hash_call(xs, ys, zs)
    g_flat = _gather_call(pt_flat, idx.reshape(NIDX))
    g = g_flat.reshape(NUM_LEVELS * 8, R, CB)
    feats = _blend_call(xs, ys, zs, g)
    feats2d = feats.reshape(2 * NUM_LEVELS, N_POINTS)
    sel2d = fsel.reshape(NBLK, 1, PTS_PER_BLK)
    out = _mlp_call(feats2d, sel2d, w1t, w2r)
    return out.reshape(N_POINTS, 1)


# speedup vs baseline: 694.9316x; 1.1557x over previous
import functools

import jax
import jax.numpy as jnp
import numpy as np
from jax import lax
from jax.experimental import pallas as pl
from jax.experimental.pallas import tpu as pltpu
from jax.experimental.pallas import tpu_sc as plsc

NUM_LEVELS = 8
BASE_RES = 16
MAX_RES = 1024
LOG2_T = 18
T = 2 ** LOG2_T
BOUND = 2.0
N_POINTS = 1048576
HIDDEN = 64
GROWTH = np.exp((np.log(MAX_RES) - np.log(BASE_RES)) / (NUM_LEVELS - 1))
RESOLUTIONS = [float(np.floor(BASE_RES * GROWTH ** l)) for l in range(NUM_LEVELS)]
P1_I32 = np.int32(np.uint32(2654435761).view(np.int32))
P2_I32 = np.int32(np.uint32(805459861).view(np.int32))

R = 1024
CB = 1024
SB = 8
NBLK = R // SB
PTS_PER_BLK = SB * CB
NIDX = NUM_LEVELS * 8 * N_POINTS

SC_WORKERS = 32
CHUNK = 16384
HALF_T = 4 * T
STAGE_W = HALF_T // 16

NSPLIT = 2
RSPLIT = R // NSPLIT


def _hash_body(xs_ref, ys_ref, zs_ref, idx_ref, sel_ref):
    x = xs_ref[...]
    y = ys_ref[...]
    z = zs_ref[...]
    inv = 1.0 / (2.0 * BOUND)
    px = (x + BOUND) * inv
    py = (y + BOUND) * inv
    pz = (z + BOUND) * inv
    sel = ((px >= 0.0) & (px <= 1.0) & (py >= 0.0) & (py <= 1.0)
           & (pz >= 0.0) & (pz <= 1.0))
    fsel = sel.astype(jnp.float32)
    px = px * fsel
    py = py * fsel
    pz = pz * fsel
    for l in range(NUM_LEVELS):
        res = np.float32(RESOLUTIONS[l])
        xi = jnp.floor(px * res).astype(jnp.int32)
        yi = jnp.floor(py * res).astype(jnp.int32)
        zi = jnp.floor(pz * res).astype(jnp.int32)
        hx = (xi, xi + 1)
        hy0 = yi * P1_I32
        hy = (hy0, hy0 + P1_I32)
        hz0 = zi * P2_I32
        hz = (hz0, hz0 + P2_I32)
        base = np.int32((l % 4) * T)
        for c in range(8):
            h = hx[c & 1] ^ hy[(c >> 1) & 1] ^ hz[(c >> 2) & 1]
            idx_ref[l * 8 + c] = (h & np.int32(T - 1)) | base
    sel_ref[...] = fsel


def _make_gather_body(per_w, nchunk):
    return functools.partial(_gather_kernel_body, per_w, nchunk)


def _gather_kernel_body(per_w, nchunk, pt_ref, idx_ref, g_ref, tab_sp,
                        idx_v0, idx_v1, g_v0, g_v1,
                        si0, si1, sg0, sg1, so0, so1):
    cid = lax.axis_index("c")
    sid = lax.axis_index("s")
    so = pl.multiple_of(sid * STAGE_W, 8)
    src = pl.multiple_of(cid * HALF_T + so, 8)
    pltpu.sync_copy(pt_ref.at[pl.ds(src, STAGE_W)], tab_sp.at[pl.ds(so, STAGE_W)])
    plsc.subcore_barrier()

    base = (cid * 16 + sid) * per_w
    npair = nchunk // 2

    def offs(jj):
        o0 = pl.multiple_of(base + (2 * jj) * CHUNK, 8)
        return o0, pl.multiple_of(base + (2 * jj + 1) * CHUNK, 8)

    o0, o1 = offs(0)
    pltpu.async_copy(idx_ref.at[pl.ds(o0, CHUNK)], idx_v0, si0)
    pltpu.async_copy(idx_ref.at[pl.ds(o1, CHUNK)], idx_v1, si1)

    @pl.loop(0, npair)
    def _pair(jj):
        o0, o1 = offs(jj)

        @pl.when(jj > 0)
        def _():
            pltpu.make_async_copy(g_v0, g_ref.at[pl.ds(o0, CHUNK)], so0).wait()
            pltpu.make_async_copy(g_v1, g_ref.at[pl.ds(o1, CHUNK)], so1).wait()

        pltpu.make_async_copy(idx_ref.at[pl.ds(o0, CHUNK)], idx_v0, si0).wait()
        pltpu.async_copy(tab_sp.at[idx_v0], g_v0, sg0)
        pltpu.make_async_copy(idx_ref.at[pl.ds(o1, CHUNK)], idx_v1, si1).wait()
        pltpu.async_copy(tab_sp.at[idx_v1], g_v1, sg1)

        pltpu.make_async_copy(tab_sp.at[idx_v0], g_v0, sg0).wait()
        pltpu.async_copy(g_v0, g_ref.at[pl.ds(o0, CHUNK)], so0)
        pltpu.make_async_copy(tab_sp.at[idx_v1], g_v1, sg1).wait()
        pltpu.async_copy(g_v1, g_ref.at[pl.ds(o1, CHUNK)], so1)

        @pl.when(jj < npair - 1)
        def _():
            n0 = pl.multiple_of(base + (2 * jj + 2) * CHUNK, 8)
            n1 = pl.multiple_of(base + (2 * jj + 3) * CHUNK, 8)
            pltpu.async_copy(idx_ref.at[pl.ds(n0, CHUNK)], idx_v0, si0)
            pltpu.async_copy(idx_ref.at[pl.ds(n1, CHUNK)], idx_v1, si1)

    oL0, oL1 = offs(npair - 1)
    pltpu.make_async_copy(g_v0, g_ref.at[pl.ds(oL0, CHUNK)], so0).wait()
    pltpu.make_async_copy(g_v1, g_ref.at[pl.ds(oL1, CHUNK)], so1).wait()


def _blend_body(xs_ref, ys_ref, zs_ref, g_ref, feats_ref):
    x = xs_ref[...]
    y = ys_ref[...]
    z = zs_ref[...]
    inv = 1.0 / (2.0 * BOUND)
    px = (x + BOUND) * inv
    py = (y + BOUND) * inv
    pz = (z + BOUND) * inv
    sel = ((px >= 0.0) & (px <= 1.0) & (py >= 0.0) & (py <= 1.0)
           & (pz >= 0.0) & (pz <= 1.0))
    fsel = sel.astype(jnp.float32)
    px = px * fsel
    py = py * fsel
    pz = pz * fsel
    for l in range(NUM_LEVELS):
        res = np.float32(RESOLUTIONS[l])
        fx = px * res
        fy = py * res
        fz = pz * res
        wx = fx - jnp.floor(fx)
        wy = fy - jnp.floor(fy)
        wz = fz - jnp.floor(fz)
        wxs = (1.0 - wx, wx)
        wys = (1.0 - wy, wy)
        wzs = (1.0 - wz, wz)
        f0 = jnp.zeros((SB, CB), jnp.float32)
        f1 = jnp.zeros((SB, CB), jnp.float32)
        for cz in range(2):
            for cy in range(2):
                wyz = wys[cy] * wzs[cz]
                for cx in range(2):
                    c = cx | (cy << 1) | (cz << 2)
                    g = g_ref[l * 8 + c]
                    a0 = lax.bitcast_convert_type(g << 16, jnp.float32)
                    a1 = lax.bitcast_convert_type(g & np.int32(-65536),
                                                  jnp.float32)
                    wc = wxs[cx] * wyz
                    f0 = f0 + wc * a0
                    f1 = f1 + wc * a1
        feats_ref[2 * l] = f0
        feats_ref[2 * l + 1] = f1


def _mlp_body(f_ref, sel_ref, w1t_ref, w2_ref, out_ref):
    fb = f_ref[...].astype(jnp.bfloat16)
    w1t = w1t_ref[...]
    h = lax.dot_general(w1t, fb, (((1,), (0,)), ((), ())),
                        preferred_element_type=jnp.float32)
    h = jnp.maximum(h, 0.0).astype(jnp.bfloat16)
    w2 = w2_ref[...]
    raw = lax.dot_general(w2, h, (((1,), (0,)), ((), ())),
                          preferred_element_type=jnp.float32)
    out_ref[...] = (jnp.exp(raw) * sel_ref[0])[None]


def _hash_call(xs, ys, zs):
    rows = xs.shape[0]
    nblk = rows // SB
    return pl.pallas_call(
        _hash_body,
        grid=(nblk,),
        in_specs=[
            pl.BlockSpec((SB, CB), lambda i: (i, 0)),
            pl.BlockSpec((SB, CB), lambda i: (i, 0)),
            pl.BlockSpec((SB, CB), lambda i: (i, 0)),
        ],
        out_specs=[
            pl.BlockSpec((NUM_LEVELS * 8, SB, CB), lambda i: (0, i, 0)),
            pl.BlockSpec((SB, CB), lambda i: (i, 0)),
        ],
        out_shape=[
            jax.ShapeDtypeStruct((NUM_LEVELS * 8, rows, CB), jnp.int32),
            jax.ShapeDtypeStruct((rows, CB), jnp.float32),
        ],
    )(xs, ys, zs)


def _gather_call(pt_flat, idx_flat):
    nidx = idx_flat.shape[0]
    per_w = nidx // SC_WORKERS
    nchunk = per_w // CHUNK
    mesh = plsc.VectorSubcoreMesh(core_axis_name="c", subcore_axis_name="s")
    kern = functools.partial(
        pl.kernel,
        out_type=jax.ShapeDtypeStruct((nidx,), jnp.int32),
        mesh=mesh,
        scratch_types=[
            pltpu.VMEM_SHARED((HALF_T,), jnp.int32),
            pltpu.VMEM((CHUNK,), jnp.int32),
            pltpu.VMEM((CHUNK,), jnp.int32),
            pltpu.VMEM((CHUNK,), jnp.int32),
            pltpu.VMEM((CHUNK,), jnp.int32),
            pltpu.SemaphoreType.DMA,
            pltpu.SemaphoreType.DMA,
            pltpu.SemaphoreType.DMA,
            pltpu.SemaphoreType.DMA,
            pltpu.SemaphoreType.DMA,
            pltpu.SemaphoreType.DMA,
        ],
    )(_make_gather_body(per_w, nchunk))
    return kern(pt_flat, idx_flat)


def _blend_call(xs, ys, zs, g):
    rows = xs.shape[0]
    nblk = rows // SB
    return pl.pallas_call(
        _blend_body,
        grid=(nblk,),
        in_specs=[
            pl.BlockSpec((SB, CB), lambda i: (i, 0)),
            pl.BlockSpec((SB, CB), lambda i: (i, 0)),
            pl.BlockSpec((SB, CB), lambda i: (i, 0)),
            pl.BlockSpec((NUM_LEVELS * 8, SB, CB), lambda i: (0, i, 0)),
        ],
        out_specs=pl.BlockSpec((2 * NUM_LEVELS, SB, CB), lambda i: (0, i, 0)),
        out_shape=jax.ShapeDtypeStruct((2 * NUM_LEVELS, rows, CB), jnp.float32),
    )(xs, ys, zs, g)


def _mlp_call(feats2d, sel2d, w1t, w2r):
    nblk = feats2d.shape[1] // PTS_PER_BLK
    return pl.pallas_call(
        _mlp_body,
        grid=(nblk,),
        in_specs=[
            pl.BlockSpec((2 * NUM_LEVELS, PTS_PER_BLK), lambda i: (0, i)),
            pl.BlockSpec((1, 1, PTS_PER_BLK), lambda i: (i, 0, 0)),
            pl.BlockSpec((HIDDEN, 2 * NUM_LEVELS), lambda i: (0, 0)),
            pl.BlockSpec((1, HIDDEN), lambda i: (0, 0)),
        ],
        out_specs=pl.BlockSpec((1, 1, PTS_PER_BLK), lambda i: (i, 0, 0)),
        out_shape=jax.ShapeDtypeStruct((nblk, 1, PTS_PER_BLK), jnp.float32),
    )(feats2d, sel2d, w1t, w2r)


def kernel(positions, viewdirs, embedded_appearance, embedded_transient,
           tables, W1, W2):
    pos_t = positions.T.reshape(3, R, CB)
    tb = lax.bitcast_convert_type(tables.astype(jnp.bfloat16), jnp.uint16)
    pt = (tb[..., 0].astype(jnp.uint32)
          | (tb[..., 1].astype(jnp.uint32) << 16))
    pt_flat = lax.bitcast_convert_type(pt, jnp.int32).reshape(NUM_LEVELS * T)
    w1t = W1.T.astype(jnp.bfloat16)
    w2r = W2.reshape(1, HIDDEN).astype(jnp.bfloat16)

    outs = []
    for s in range(NSPLIT):
        r0, r1 = s * RSPLIT, (s + 1) * RSPLIT
        xs, ys, zs = pos_t[0, r0:r1], pos_t[1, r0:r1], pos_t[2, r0:r1]
        npts = RSPLIT * CB
        nidx = NUM_LEVELS * 8 * npts
        idx, fsel = _hash_call(xs, ys, zs)
        g_flat = _gather_call(pt_flat, idx.reshape(nidx))
        g = g_flat.reshape(NUM_LEVELS * 8, RSPLIT, CB)
        feats = _blend_call(xs, ys, zs, g)
        feats2d = feats.reshape(2 * NUM_LEVELS, npts)
        sel2d = fsel.reshape(npts // PTS_PER_BLK, 1, PTS_PER_BLK)
        out = _mlp_call(feats2d, sel2d, w1t, w2r)
        outs.append(out.reshape(npts))
    return jnp.concatenate(outs).reshape(N_POINTS, 1)


# speedup vs baseline: 725.3727x; 1.0438x over previous
import functools

import jax
import jax.numpy as jnp
import numpy as np
from jax import lax
from jax.experimental import pallas as pl
from jax.experimental.pallas import tpu as pltpu
from jax.experimental.pallas import tpu_sc as plsc

NUM_LEVELS = 8
BASE_RES = 16
MAX_RES = 1024
LOG2_T = 18
T = 2 ** LOG2_T
BOUND = 2.0
N_POINTS = 1048576
HIDDEN = 64
GROWTH = np.exp((np.log(MAX_RES) - np.log(BASE_RES)) / (NUM_LEVELS - 1))
RESOLUTIONS = [float(np.floor(BASE_RES * GROWTH ** l)) for l in range(NUM_LEVELS)]
P1_I32 = np.int32(np.uint32(2654435761).view(np.int32))
P2_I32 = np.int32(np.uint32(805459861).view(np.int32))

R = 1024
CB = 1024
SB = 8
NBLK = R // SB
PTS_PER_BLK = SB * CB
NIDX = NUM_LEVELS * 8 * N_POINTS

SC_WORKERS = 32
CHUNK = 16384
HALF_T = 4 * T
STAGE_W = HALF_T // 16

NSPLIT = 4
RSPLIT = R // NSPLIT


def _hash_body(xs_ref, ys_ref, zs_ref, idx_ref, sel_ref):
    x = xs_ref[...]
    y = ys_ref[...]
    z = zs_ref[...]
    inv = 1.0 / (2.0 * BOUND)
    px = (x + BOUND) * inv
    py = (y + BOUND) * inv
    pz = (z + BOUND) * inv
    sel = ((px >= 0.0) & (px <= 1.0) & (py >= 0.0) & (py <= 1.0)
           & (pz >= 0.0) & (pz <= 1.0))
    fsel = sel.astype(jnp.float32)
    px = px * fsel
    py = py * fsel
    pz = pz * fsel
    for l in range(NUM_LEVELS):
        res = np.float32(RESOLUTIONS[l])
        xi = jnp.floor(px * res).astype(jnp.int32)
        yi = jnp.floor(py * res).astype(jnp.int32)
        zi = jnp.floor(pz * res).astype(jnp.int32)
        hx = (xi, xi + 1)
        hy0 = yi * P1_I32
        hy = (hy0, hy0 + P1_I32)
        hz0 = zi * P2_I32
        hz = (hz0, hz0 + P2_I32)
        base = np.int32((l % 4) * T)
        for c in range(8):
            h = hx[c & 1] ^ hy[(c >> 1) & 1] ^ hz[(c >> 2) & 1]
            idx_ref[l * 8 + c] = (h & np.int32(T - 1)) | base
    sel_ref[...] = fsel


def _make_gather_body(per_w, nchunk):
    return functools.partial(_gather_kernel_body, per_w, nchunk)


def _gather_kernel_body(per_w, nchunk, pt_ref, idx_ref, g_ref, tab_sp,
                        idx_v0, idx_v1, g_v0, g_v1,
                        si0, si1, sg0, sg1, so0, so1):
    cid = lax.axis_index("c")
    sid = lax.axis_index("s")
    so = pl.multiple_of(sid * STAGE_W, 8)
    src = pl.multiple_of(cid * HALF_T + so, 8)
    pltpu.sync_copy(pt_ref.at[pl.ds(src, STAGE_W)], tab_sp.at[pl.ds(so, STAGE_W)])
    plsc.subcore_barrier()

    base = (cid * 16 + sid) * per_w
    npair = nchunk // 2

    def offs(jj):
        o0 = pl.multiple_of(base + (2 * jj) * CHUNK, 8)
        return o0, pl.multiple_of(base + (2 * jj + 1) * CHUNK, 8)

    o0, o1 = offs(0)
    pltpu.async_copy(idx_ref.at[pl.ds(o0, CHUNK)], idx_v0, si0)
    pltpu.async_copy(idx_ref.at[pl.ds(o1, CHUNK)], idx_v1, si1)

    @pl.loop(0, npair)
    def _pair(jj):
        o0, o1 = offs(jj)

        @pl.when(jj > 0)
        def _():
            pltpu.make_async_copy(g_v0, g_ref.at[pl.ds(o0, CHUNK)], so0).wait()
            pltpu.make_async_copy(g_v1, g_ref.at[pl.ds(o1, CHUNK)], so1).wait()

        pltpu.make_async_copy(idx_ref.at[pl.ds(o0, CHUNK)], idx_v0, si0).wait()
        pltpu.async_copy(tab_sp.at[idx_v0], g_v0, sg0)
        pltpu.make_async_copy(idx_ref.at[pl.ds(o1, CHUNK)], idx_v1, si1).wait()
        pltpu.async_copy(tab_sp.at[idx_v1], g_v1, sg1)

        pltpu.make_async_copy(tab_sp.at[idx_v0], g_v0, sg0).wait()
        pltpu.async_copy(g_v0, g_ref.at[pl.ds(o0, CHUNK)], so0)
        pltpu.make_async_copy(tab_sp.at[idx_v1], g_v1, sg1).wait()
        pltpu.async_copy(g_v1, g_ref.at[pl.ds(o1, CHUNK)], so1)

        @pl.when(jj < npair - 1)
        def _():
            n0 = pl.multiple_of(base + (2 * jj + 2) * CHUNK, 8)
            n1 = pl.multiple_of(base + (2 * jj + 3) * CHUNK, 8)
            pltpu.async_copy(idx_ref.at[pl.ds(n0, CHUNK)], idx_v0, si0)
            pltpu.async_copy(idx_ref.at[pl.ds(n1, CHUNK)], idx_v1, si1)

    oL0, oL1 = offs(npair - 1)
    pltpu.make_async_copy(g_v0, g_ref.at[pl.ds(oL0, CHUNK)], so0).wait()
    pltpu.make_async_copy(g_v1, g_ref.at[pl.ds(oL1, CHUNK)], so1).wait()


def _blend_body(xs_ref, ys_ref, zs_ref, g_ref, feats_ref):
    x = xs_ref[...]
    y = ys_ref[...]
    z = zs_ref[...]
    inv = 1.0 / (2.0 * BOUND)
    px = (x + BOUND) * inv
    py = (y + BOUND) * inv
    pz = (z + BOUND) * inv
    sel = ((px >= 0.0) & (px <= 1.0) & (py >= 0.0) & (py <= 1.0)
           & (pz >= 0.0) & (pz <= 1.0))
    fsel = sel.astype(jnp.float32)
    px = px * fsel
    py = py * fsel
    pz = pz * fsel
    for l in range(NUM_LEVELS):
        res = np.float32(RESOLUTIONS[l])
        fx = px * res
        fy = py * res
        fz = pz * res
        wx = fx - jnp.floor(fx)
        wy = fy - jnp.floor(fy)
        wz = fz - jnp.floor(fz)
        wxs = (1.0 - wx, wx)
        wys = (1.0 - wy, wy)
        wzs = (1.0 - wz, wz)
        f0 = jnp.zeros((SB, CB), jnp.float32)
        f1 = jnp.zeros((SB, CB), jnp.float32)
        for cz in range(2):
            for cy in range(2):
                wyz = wys[cy] * wzs[cz]
                for cx in range(2):
                    c = cx | (cy << 1) | (cz << 2)
                    g = g_ref[l * 8 + c]
                    a0 = lax.bitcast_convert_type(g << 16, jnp.float32)
                    a1 = lax.bitcast_convert_type(g & np.int32(-65536),
                                                  jnp.float32)
                    wc = wxs[cx] * wyz
                    f0 = f0 + wc * a0
                    f1 = f1 + wc * a1
        feats_ref[2 * l] = f0
        feats_ref[2 * l + 1] = f1


def _mlp_body(f_ref, sel_ref, w1t_ref, w2_ref, out_ref):
    fb = f_ref[...].astype(jnp.bfloat16)
    w1t = w1t_ref[...]
    h = lax.dot_general(w1t, fb, (((1,), (0,)), ((), ())),
                        preferred_element_type=jnp.float32)
    h = jnp.maximum(h, 0.0).astype(jnp.bfloat16)
    w2 = w2_ref[...]
    raw = lax.dot_general(w2, h, (((1,), (0,)), ((), ())),
                          preferred_element_type=jnp.float32)
    out_ref[...] = (jnp.exp(raw) * sel_ref[0])[None]


def _hash_call(xs, ys, zs):
    rows = xs.shape[0]
    nblk = rows // SB
    return pl.pallas_call(
        _hash_body,
        grid=(nblk,),
        in_specs=[
            pl.BlockSpec((SB, CB), lambda i: (i, 0)),
            pl.BlockSpec((SB, CB), lambda i: (i, 0)),
            pl.BlockSpec((SB, CB), lambda i: (i, 0)),
        ],
        out_specs=[
            pl.BlockSpec((NUM_LEVELS * 8, SB, CB), lambda i: (0, i, 0)),
            pl.BlockSpec((SB, CB), lambda i: (i, 0)),
        ],
        out_shape=[
            jax.ShapeDtypeStruct((NUM_LEVELS * 8, rows, CB), jnp.int32),
            jax.ShapeDtypeStruct((rows, CB), jnp.float32),
        ],
    )(xs, ys, zs)


def _gather_call(pt_flat, idx_flat):
    nidx = idx_flat.shape[0]
    per_w = nidx // SC_WORKERS
    nchunk = per_w // CHUNK
    mesh = plsc.VectorSubcoreMesh(core_axis_name="c", subcore_axis_name="s")
    kern = functools.partial(
        pl.kernel,
        out_type=jax.ShapeDtypeStruct((nidx,), jnp.int32),
        mesh=mesh,
        scratch_types=[
            pltpu.VMEM_SHARED((HALF_T,), jnp.int32),
            pltpu.VMEM((CHUNK,), jnp.int32),
            pltpu.VMEM((CHUNK,), jnp.int32),
            pltpu.VMEM((CHUNK,), jnp.int32),
            pltpu.VMEM((CHUNK,), jnp.int32),
            pltpu.SemaphoreType.DMA,
            pltpu.SemaphoreType.DMA,
            pltpu.SemaphoreType.DMA,
            pltpu.SemaphoreType.DMA,
            pltpu.SemaphoreType.DMA,
            pltpu.SemaphoreType.DMA,
        ],
    )(_make_gather_body(per_w, nchunk))
    return kern(pt_flat, idx_flat)


def _blend_call(xs, ys, zs, g):
    rows = xs.shape[0]
    nblk = rows // SB
    return pl.pallas_call(
        _blend_body,
        grid=(nblk,),
        in_specs=[
            pl.BlockSpec((SB, CB), lambda i: (i, 0)),
            pl.BlockSpec((SB, CB), lambda i: (i, 0)),
            pl.BlockSpec((SB, CB), lambda i: (i, 0)),
            pl.BlockSpec((NUM_LEVELS * 8, SB, CB), lambda i: (0, i, 0)),
        ],
        out_specs=pl.BlockSpec((2 * NUM_LEVELS, SB, CB), lambda i: (0, i, 0)),
        out_shape=jax.ShapeDtypeStruct((2 * NUM_LEVELS, rows, CB), jnp.float32),
    )(xs, ys, zs, g)


def _mlp_call(feats2d, sel2d, w1t, w2r):
    nblk = feats2d.shape[1] // PTS_PER_BLK
    return pl.pallas_call(
        _mlp_body,
        grid=(nblk,),
        in_specs=[
            pl.BlockSpec((2 * NUM_LEVELS, PTS_PER_BLK), lambda i: (0, i)),
            pl.BlockSpec((1, 1, PTS_PER_BLK), lambda i: (i, 0, 0)),
            pl.BlockSpec((HIDDEN, 2 * NUM_LEVELS), lambda i: (0, 0)),
            pl.BlockSpec((1, HIDDEN), lambda i: (0, 0)),
        ],
        out_specs=pl.BlockSpec((1, 1, PTS_PER_BLK), lambda i: (i, 0, 0)),
        out_shape=jax.ShapeDtypeStruct((nblk, 1, PTS_PER_BLK), jnp.float32),
    )(feats2d, sel2d, w1t, w2r)


def kernel(positions, viewdirs, embedded_appearance, embedded_transient,
           tables, W1, W2):
    pos_t = positions.T.reshape(3, R, CB)
    tb = lax.bitcast_convert_type(tables.astype(jnp.bfloat16), jnp.uint16)
    pt = (tb[..., 0].astype(jnp.uint32)
          | (tb[..., 1].astype(jnp.uint32) << 16))
    pt_flat = lax.bitcast_convert_type(pt, jnp.int32).reshape(NUM_LEVELS * T)
    w1t = W1.T.astype(jnp.bfloat16)
    w2r = W2.reshape(1, HIDDEN).astype(jnp.bfloat16)

    outs = []
    for s in range(NSPLIT):
        r0, r1 = s * RSPLIT, (s + 1) * RSPLIT
        xs, ys, zs = pos_t[0, r0:r1], pos_t[1, r0:r1], pos_t[2, r0:r1]
        npts = RSPLIT * CB
        nidx = NUM_LEVELS * 8 * npts
        idx, fsel = _hash_call(xs, ys, zs)
        g_flat = _gather_call(pt_flat, idx.reshape(nidx))
        g = g_flat.reshape(NUM_LEVELS * 8, RSPLIT, CB)
        feats = _blend_call(xs, ys, zs, g)
        feats2d = feats.reshape(2 * NUM_LEVELS, npts)
        sel2d = fsel.reshape(npts // PTS_PER_BLK, 1, PTS_PER_BLK)
        out = _mlp_call(feats2d, sel2d, w1t, w2r)
        outs.append(out.reshape(npts))
    return jnp.concatenate(outs).reshape(N_POINTS, 1)


# speedup vs baseline: 726.1965x; 1.0011x over previous
import functools

import jax
import jax.numpy as jnp
import numpy as np
from jax import lax
from jax.experimental import pallas as pl
from jax.experimental.pallas import tpu as pltpu
from jax.experimental.pallas import tpu_sc as plsc

NUM_LEVELS = 8
BASE_RES = 16
MAX_RES = 1024
LOG2_T = 18
T = 2 ** LOG2_T
BOUND = 2.0
N_POINTS = 1048576
HIDDEN = 64
GROWTH = np.exp((np.log(MAX_RES) - np.log(BASE_RES)) / (NUM_LEVELS - 1))
RESOLUTIONS = [float(np.floor(BASE_RES * GROWTH ** l)) for l in range(NUM_LEVELS)]
P1_I32 = np.int32(np.uint32(2654435761).view(np.int32))
P2_I32 = np.int32(np.uint32(805459861).view(np.int32))

R = 1024
CB = 1024
SB = 8
NBLK = R // SB
PTS_PER_BLK = SB * CB
NIDX = NUM_LEVELS * 8 * N_POINTS

SC_WORKERS = 32
CHUNK = 16384
HALF_T = 4 * T
STAGE_W = HALF_T // 16

NSPLIT = 4
RSPLIT = R // NSPLIT


def _hash_body(xs_ref, ys_ref, zs_ref, idx_ref, sel_ref):
    x = xs_ref[...]
    y = ys_ref[...]
    z = zs_ref[...]
    inv = 1.0 / (2.0 * BOUND)
    px = (x + BOUND) * inv
    py = (y + BOUND) * inv
    pz = (z + BOUND) * inv
    sel = ((px >= 0.0) & (px <= 1.0) & (py >= 0.0) & (py <= 1.0)
           & (pz >= 0.0) & (pz <= 1.0))
    fsel = sel.astype(jnp.float32)
    px = px * fsel
    py = py * fsel
    pz = pz * fsel
    for l in range(NUM_LEVELS):
        res = np.float32(RESOLUTIONS[l])
        xi = jnp.floor(px * res).astype(jnp.int32)
        yi = jnp.floor(py * res).astype(jnp.int32)
        zi = jnp.floor(pz * res).astype(jnp.int32)
        hx = (xi, xi + 1)
        hy0 = yi * P1_I32
        hy = (hy0, hy0 + P1_I32)
        hz0 = zi * P2_I32
        hz = (hz0, hz0 + P2_I32)
        base = np.int32((l % 4) * T)
        for c in range(8):
            h = hx[c & 1] ^ hy[(c >> 1) & 1] ^ hz[(c >> 2) & 1]
            idx_ref[l * 8 + c] = (h & np.int32(T - 1)) | base
    sel_ref[...] = fsel


def _make_gather_body(per_w, nchunk):
    return functools.partial(_gather_kernel_body, per_w, nchunk)


def _gather_kernel_body(per_w, nchunk, pt_ref, idx_ref, g_ref, tab_sp,
                        idx_v0, idx_v1, g_v0, g_v1,
                        si0, si1, sg0, sg1, so0, so1):
    cid = lax.axis_index("c")
    sid = lax.axis_index("s")
    so = pl.multiple_of(sid * STAGE_W, 8)
    src = pl.multiple_of(cid * HALF_T + so, 8)
    pltpu.sync_copy(pt_ref.at[pl.ds(src, STAGE_W)], tab_sp.at[pl.ds(so, STAGE_W)])
    plsc.subcore_barrier()

    base = (cid * 16 + sid) * per_w
    npair = nchunk // 2

    def offs(jj):
        o0 = pl.multiple_of(base + (2 * jj) * CHUNK, 8)
        return o0, pl.multiple_of(base + (2 * jj + 1) * CHUNK, 8)

    o0, o1 = offs(0)
    pltpu.async_copy(idx_ref.at[pl.ds(o0, CHUNK)], idx_v0, si0)
    pltpu.async_copy(idx_ref.at[pl.ds(o1, CHUNK)], idx_v1, si1)

    @pl.loop(0, npair)
    def _pair(jj):
        o0, o1 = offs(jj)

        @pl.when(jj > 0)
        def _():
            pltpu.make_async_copy(g_v0, g_ref.at[pl.ds(o0, CHUNK)], so0).wait()
            pltpu.make_async_copy(g_v1, g_ref.at[pl.ds(o1, CHUNK)], so1).wait()

        pltpu.make_async_copy(idx_ref.at[pl.ds(o0, CHUNK)], idx_v0, si0).wait()
        pltpu.async_copy(tab_sp.at[idx_v0], g_v0, sg0)
        pltpu.make_async_copy(idx_ref.at[pl.ds(o1, CHUNK)], idx_v1, si1).wait()
        pltpu.async_copy(tab_sp.at[idx_v1], g_v1, sg1)

        pltpu.make_async_copy(tab_sp.at[idx_v0], g_v0, sg0).wait()
        pltpu.async_copy(g_v0, g_ref.at[pl.ds(o0, CHUNK)], so0)
        pltpu.make_async_copy(tab_sp.at[idx_v1], g_v1, sg1).wait()
        pltpu.async_copy(g_v1, g_ref.at[pl.ds(o1, CHUNK)], so1)

        @pl.when(jj < npair - 1)
        def _():
            n0 = pl.multiple_of(base + (2 * jj + 2) * CHUNK, 8)
            n1 = pl.multiple_of(base + (2 * jj + 3) * CHUNK, 8)
            pltpu.async_copy(idx_ref.at[pl.ds(n0, CHUNK)], idx_v0, si0)
            pltpu.async_copy(idx_ref.at[pl.ds(n1, CHUNK)], idx_v1, si1)

    oL0, oL1 = offs(npair - 1)
    pltpu.make_async_copy(g_v0, g_ref.at[pl.ds(oL0, CHUNK)], so0).wait()
    pltpu.make_async_copy(g_v1, g_ref.at[pl.ds(oL1, CHUNK)], so1).wait()


def _blend_body(xs_ref, ys_ref, zs_ref, g_ref, feats_ref):
    x = xs_ref[...]
    y = ys_ref[...]
    z = zs_ref[...]
    inv = 1.0 / (2.0 * BOUND)
    px = (x + BOUND) * inv
    py = (y + BOUND) * inv
    pz = (z + BOUND) * inv
    sel = ((px >= 0.0) & (px <= 1.0) & (py >= 0.0) & (py <= 1.0)
           & (pz >= 0.0) & (pz <= 1.0))
    fsel = sel.astype(jnp.float32)
    px = px * fsel
    py = py * fsel
    pz = pz * fsel
    for l in range(NUM_LEVELS):
        res = np.float32(RESOLUTIONS[l])
        fx = px * res
        fy = py * res
        fz = pz * res
        wx = fx - jnp.floor(fx)
        wy = fy - jnp.floor(fy)
        wz = fz - jnp.floor(fz)
        wxs = (1.0 - wx, wx)
        wys = (1.0 - wy, wy)
        wzs = (1.0 - wz, wz)
        f0 = jnp.zeros((SB, CB), jnp.float32)
        f1 = jnp.zeros((SB, CB), jnp.float32)
        for cz in range(2):
            for cy in range(2):
                wyz = wys[cy] * wzs[cz]
                for cx in range(2):
                    c = cx | (cy << 1) | (cz << 2)
                    g = g_ref[l * 8 + c]
                    a0 = lax.bitcast_convert_type(g << 16, jnp.float32)
                    a1 = lax.bitcast_convert_type(g & np.int32(-65536),
                                                  jnp.float32)
                    wc = wxs[cx] * wyz
                    f0 = f0 + wc * a0
                    f1 = f1 + wc * a1
        feats_ref[2 * l] = f0
        feats_ref[2 * l + 1] = f1


def _mlp_body(f_ref, sel_ref, w1t_ref, w2_ref, out_ref):
    fb = f_ref[...].astype(jnp.bfloat16)
    w1t = w1t_ref[...]
    h = lax.dot_general(w1t, fb, (((1,), (0,)), ((), ())),
                        preferred_element_type=jnp.float32)
    h = jnp.maximum(h, 0.0).astype(jnp.bfloat16)
    w2 = w2_ref[...]
    raw = lax.dot_general(w2, h, (((1,), (0,)), ((), ())),
                          preferred_element_type=jnp.float32)
    out_ref[...] = (jnp.exp(raw) * sel_ref[0])[None]


def _hash_call(xs, ys, zs):
    rows = xs.shape[0]
    nblk = rows // SB
    return pl.pallas_call(
        _hash_body,
        grid=(nblk,),
        in_specs=[
            pl.BlockSpec((SB, CB), lambda i: (i, 0)),
            pl.BlockSpec((SB, CB), lambda i: (i, 0)),
            pl.BlockSpec((SB, CB), lambda i: (i, 0)),
        ],
        out_specs=[
            pl.BlockSpec((NUM_LEVELS * 8, SB, CB), lambda i: (0, i, 0)),
            pl.BlockSpec((SB, CB), lambda i: (i, 0)),
        ],
        out_shape=[
            jax.ShapeDtypeStruct((NUM_LEVELS * 8, rows, CB), jnp.int32),
            jax.ShapeDtypeStruct((rows, CB), jnp.float32),
        ],
    )(xs, ys, zs)


def _gather_call(pt_flat, idx_flat):
    nidx = idx_flat.shape[0]
    per_w = nidx // SC_WORKERS
    nchunk = per_w // CHUNK
    mesh = plsc.VectorSubcoreMesh(core_axis_name="c", subcore_axis_name="s")
    kern = functools.partial(
        pl.kernel,
        out_type=jax.ShapeDtypeStruct((nidx,), jnp.int32),
        mesh=mesh,
        compiler_params=pltpu.CompilerParams(use_tc_tiling_on_sc=True),
        scratch_types=[
            pltpu.VMEM_SHARED((HALF_T,), jnp.int32),
            pltpu.VMEM((CHUNK,), jnp.int32),
            pltpu.VMEM((CHUNK,), jnp.int32),
            pltpu.VMEM((CHUNK,), jnp.int32),
            pltpu.VMEM((CHUNK,), jnp.int32),
            pltpu.SemaphoreType.DMA,
            pltpu.SemaphoreType.DMA,
            pltpu.SemaphoreType.DMA,
            pltpu.SemaphoreType.DMA,
            pltpu.SemaphoreType.DMA,
            pltpu.SemaphoreType.DMA,
        ],
    )(_make_gather_body(per_w, nchunk))
    return kern(pt_flat, idx_flat)


def _blend_call(xs, ys, zs, g):
    rows = xs.shape[0]
    nblk = rows // SB
    return pl.pallas_call(
        _blend_body,
        grid=(nblk,),
        in_specs=[
            pl.BlockSpec((SB, CB), lambda i: (i, 0)),
            pl.BlockSpec((SB, CB), lambda i: (i, 0)),
            pl.BlockSpec((SB, CB), lambda i: (i, 0)),
            pl.BlockSpec((NUM_LEVELS * 8, SB, CB), lambda i: (0, i, 0)),
        ],
        out_specs=pl.BlockSpec((2 * NUM_LEVELS, SB, CB), lambda i: (0, i, 0)),
        out_shape=jax.ShapeDtypeStruct((2 * NUM_LEVELS, rows, CB), jnp.float32),
    )(xs, ys, zs, g)


def _mlp_call(feats2d, sel2d, w1t, w2r):
    nblk = feats2d.shape[1] // PTS_PER_BLK
    return pl.pallas_call(
        _mlp_body,
        grid=(nblk,),
        in_specs=[
            pl.BlockSpec((2 * NUM_LEVELS, PTS_PER_BLK), lambda i: (0, i)),
            pl.BlockSpec((1, 1, PTS_PER_BLK), lambda i: (i, 0, 0)),
            pl.BlockSpec((HIDDEN, 2 * NUM_LEVELS), lambda i: (0, 0)),
            pl.BlockSpec((1, HIDDEN), lambda i: (0, 0)),
        ],
        out_specs=pl.BlockSpec((1, 1, PTS_PER_BLK), lambda i: (i, 0, 0)),
        out_shape=jax.ShapeDtypeStruct((nblk, 1, PTS_PER_BLK), jnp.float32),
    )(feats2d, sel2d, w1t, w2r)


def kernel(positions, viewdirs, embedded_appearance, embedded_transient,
           tables, W1, W2):
    pos_t = positions.T.reshape(3, R, CB)
    tb = lax.bitcast_convert_type(tables.astype(jnp.bfloat16), jnp.uint16)
    pt = (tb[..., 0].astype(jnp.uint32)
          | (tb[..., 1].astype(jnp.uint32) << 16))
    pt_flat = lax.bitcast_convert_type(pt, jnp.int32).reshape(NUM_LEVELS * T)
    w1t = W1.T.astype(jnp.bfloat16)
    w2r = W2.reshape(1, HIDDEN).astype(jnp.bfloat16)

    outs = []
    for s in range(NSPLIT):
        r0, r1 = s * RSPLIT, (s + 1) * RSPLIT
        xs, ys, zs = pos_t[0, r0:r1], pos_t[1, r0:r1], pos_t[2, r0:r1]
        npts = RSPLIT * CB
        nidx = NUM_LEVELS * 8 * npts
        idx, fsel = _hash_call(xs, ys, zs)
        g_flat = _gather_call(pt_flat, idx.reshape(nidx))
        g = g_flat.reshape(NUM_LEVELS * 8, RSPLIT, CB)
        feats = _blend_call(xs, ys, zs, g)
        feats2d = feats.reshape(2 * NUM_LEVELS, npts)
        sel2d = fsel.reshape(npts // PTS_PER_BLK, 1, PTS_PER_BLK)
        out = _mlp_call(feats2d, sel2d, w1t, w2r)
        outs.append(out.reshape(npts))
    return jnp.concatenate(outs).reshape(N_POINTS, 1)
